# baseline (speedup 1.0000x reference)
import dataclasses
import functools

import jax
import jax.numpy as jnp
from jax import lax
from jax.experimental import pallas as pl
from jax.experimental.pallas import tpu as pltpu
from jax.experimental.pallas import tpu_sc as plsc

N = 10000
D = 128
E = 320000

NC = 2
NS = 16
NW = NC * NS

CH = 128
NCHUNK = 80
EPT = NCHUNK * CH
E_PAD = NW * EPT
NPAD = 10240
RPT = NPAD // NS
DEGW = 128

SCH = 64
SNCHUNK = EPT // SCH
NBUF = 4
SNPASS = 4
SCPP = SNCHUNK // SNPASS

RB = 512
GRID = NPAD // RB


@functools.cache
def _sc_kernels():
    mesh = plsc.VectorSubcoreMesh(core_axis_name="c", subcore_axis_name="s")
    cp = pltpu.CompilerParams()
    if "needs_layout_passes" in pltpu.CompilerParams.__dataclass_fields__:
        cp = dataclasses.replace(cp, needs_layout_passes=False)

    L = 16
    HR = NPAD // D

    @functools.partial(
        pl.kernel,
        out_type=jax.ShapeDtypeStruct((NC, HR, D), jnp.float32),
        mesh=mesh,
        compiler_params=cp,
        scratch_types=[
            pltpu.VMEM((NCHUNK, CH), jnp.int32),
            pltpu.VMEM((HR, D), jnp.float32),
            pltpu.VMEM((HR,), jnp.int32),
            pltpu.VMEM_SHARED((HR, D), jnp.float32),
        ],
    )
    def deg_kernel(dst_hbm, zeros_hbm, rowid_hbm, out_hbm,
                   dst_v, hist_v, rowid_v, deg_sh):
        cid = lax.axis_index("c")
        sid = lax.axis_index("s")
        wid = sid * NC + cid
        pltpu.sync_copy(dst_hbm.at[wid], dst_v)
        pltpu.sync_copy(zeros_hbm, hist_v)
        pltpu.sync_copy(rowid_hbm, rowid_v)

        @pl.when(sid == 0)
        def _():
            pltpu.sync_copy(zeros_hbm, deg_sh)

        ones = jnp.ones((L,), jnp.float32)

        @pl.loop(0, NCHUNK)
        def _(j):
            @pl.loop(0, CH, step=L)
            def _(k):
                idx = dst_v.at[j][pl.ds(k, L)]
                hi = jnp.right_shift(idx, 7)
                lo = jnp.bitwise_and(idx, 127)
                plsc.addupdate_scatter(hist_v, [hi, lo], ones)

        plsc.subcore_barrier()
        pltpu.sync_copy(hist_v, deg_sh.at[rowid_v], add=True)
        plsc.subcore_barrier()

        @pl.when(sid == 0)
        def _():
            pltpu.sync_copy(deg_sh, out_hbm.at[cid])

    @functools.partial(
        pl.kernel,
        out_type=jax.ShapeDtypeStruct((NC, NPAD, D), jnp.float32),
        mesh=mesh,
        scratch_types=[
            pltpu.VMEM((SCPP, SCH), jnp.int32),
            pltpu.VMEM((SCPP, SCH), jnp.int32),
        ] + [pltpu.VMEM((SCH, D), jnp.float32)] * NBUF + [
            pltpu.VMEM_SHARED((NPAD, D), jnp.float32),
        ] + [pltpu.SemaphoreType.DMA] * (2 * NBUF),
    )
    def segsum_kernel(y_hbm, src_hbm, dst_hbm, zeros_hbm, out_hbm,
                      src_v, dst_v, *rest):
        rows = rest[:NBUF]
        acc_sh = rest[NBUF]
        gsem = rest[NBUF + 1:NBUF + 1 + NBUF]
        ssem = rest[NBUF + 1 + NBUF:]
        cid = lax.axis_index("c")
        sid = lax.axis_index("s")
        wid = sid * NC + cid
        pltpu.sync_copy(zeros_hbm, acc_sh.at[pl.ds(sid * RPT, RPT)])
        plsc.subcore_barrier()

        GH = SCH // 2

        def gather(j, b):
            for h in range(2):
                pltpu.async_copy(y_hbm.at[src_v.at[j, pl.ds(h * GH, GH)]],
                                 rows[b].at[pl.ds(h * GH, GH)], gsem[b])

        def gather_wait(j, b):
            for h in range(2):
                pltpu.make_async_copy(
                    y_hbm.at[src_v.at[j, pl.ds(h * GH, GH)]],
                    rows[b].at[pl.ds(h * GH, GH)], gsem[b]).wait()

        def scatter(j, b):
            return pltpu.async_copy(rows[b], acc_sh.at[dst_v.at[j]], ssem[b],
                                    add=True)

        def scatter_wait(j, b):
            pltpu.make_async_copy(rows[b], acc_sh.at[dst_v.at[j]],
                                  ssem[b]).wait()

        for p in range(SNPASS):
            pltpu.sync_copy(src_hbm.at[wid, pl.ds(p * SCPP, SCPP)], src_v)
            pltpu.sync_copy(dst_hbm.at[wid, pl.ds(p * SCPP, SCPP)], dst_v)
            for b in range(NBUF):
                gather(b, b)

            @pl.loop(0, SCPP, step=NBUF)
            def _(j):
                for b in range(NBUF):
                    gather_wait(j + b, b)
                    scatter(j + b, b)
                for b in range(NBUF):
                    scatter_wait(j + b, b)

                    @pl.when(j + b + NBUF < SCPP)
                    def _():
                        gather(j + b + NBUF, b)

        plsc.subcore_barrier()
        pltpu.sync_copy(
            acc_sh.at[pl.ds(sid * RPT, RPT)],
            out_hbm.at[cid, pl.ds(sid * RPT, RPT)],
        )

    return deg_kernel, segsum_kernel


def _dinv_of(deg_ref):
    deg = deg_ref[0, :, 0] + deg_ref[1, :, 0] + 1.0
    return lax.rsqrt(deg)


def _mm_scale_body(x_ref, w_ref, deg_ref, y_ref):
    dinv = _dinv_of(deg_ref)
    xw = jnp.dot(x_ref[...], w_ref[...], preferred_element_type=jnp.float32,
                 precision=lax.Precision.HIGHEST)
    y_ref[...] = dinv[:, None] * xw


def _epi_mm_body(acc_ref, y_ref, deg_ref, b_ref, w_ref, out_ref):
    dinv = _dinv_of(deg_ref)
    s = acc_ref[0] + acc_ref[1] + y_ref[...]
    z = jnp.maximum(dinv[:, None] * s + b_ref[0], 0.0)
    zw = jnp.dot(z, w_ref[...], preferred_element_type=jnp.float32,
                 precision=lax.Precision.HIGHEST)
    out_ref[...] = dinv[:, None] * zw


def _epi_final_body(acc_ref, y_ref, deg_ref, b_ref, out_ref):
    dinv = _dinv_of(deg_ref)
    s = acc_ref[0] + acc_ref[1] + y_ref[...]
    out_ref[...] = jnp.maximum(dinv[:, None] * s + b_ref[0], 0.0)


_spec_rows = pl.BlockSpec((RB, D), lambda i: (i, 0))
_spec_w = pl.BlockSpec((D, D), lambda i: (0, 0))
_spec_deg = pl.BlockSpec((NC, RB, 1), lambda i: (0, i, 0))
_spec_acc = pl.BlockSpec((NC, RB, D), lambda i: (0, i, 0))
_spec_b = pl.BlockSpec((1, D), lambda i: (0, 0))
_out_rows = jax.ShapeDtypeStruct((NPAD, D), jnp.float32)

_mm_scale = pl.pallas_call(
    _mm_scale_body, grid=(GRID,),
    in_specs=[_spec_rows, _spec_w, _spec_deg],
    out_specs=_spec_rows, out_shape=_out_rows)

_epi_mm = pl.pallas_call(
    _epi_mm_body, grid=(GRID,),
    in_specs=[_spec_acc, _spec_rows, _spec_deg, _spec_b, _spec_w],
    out_specs=_spec_rows, out_shape=_out_rows)

_epi_final = pl.pallas_call(
    _epi_final_body, grid=(GRID,),
    in_specs=[_spec_acc, _spec_rows, _spec_deg, _spec_b],
    out_specs=_spec_rows, out_shape=_out_rows)


def kernel(x, edge_index, W1, b1, W2, b2):
    src = edge_index[0].astype(jnp.int32)
    dst = edge_index[1].astype(jnp.int32)
    ppt = EPT - E // NW
    pad_src = jnp.zeros((NW, ppt), jnp.int32)
    pad_dst = jnp.broadcast_to(N + jnp.arange(ppt, dtype=jnp.int32) % (NPAD - N),
                               (NW, ppt))
    src_t = jnp.concatenate([src.reshape(NW, E // NW), pad_src], axis=1)
    dst_t = jnp.concatenate([dst.reshape(NW, E // NW), pad_dst], axis=1)
    src_r = src_t.reshape(NW, SNCHUNK, SCH)
    dst_r = dst_t.reshape(NW, SNCHUNK, SCH)
    dst_d = dst_t.reshape(NW, NCHUNK, CH)

    x_p = jnp.pad(x, ((0, NPAD - N), (0, 0)))
    zeros_h = jnp.zeros((NPAD // D, D), jnp.float32)
    rowid = jnp.arange(NPAD // D, dtype=jnp.int32)
    zeros_r = jnp.zeros((RPT, D), jnp.float32)
    b1r = b1.reshape(1, D)
    b2r = b2.reshape(1, D)

    deg_kernel, segsum_kernel = _sc_kernels()
    deg = deg_kernel(dst_d, zeros_h, rowid).reshape(NC, NPAD, 1)

    y1 = _mm_scale(x_p, W1, deg)
    acc1 = segsum_kernel(y1, src_r, dst_r, zeros_r)
    y2 = _epi_mm(acc1, y1, deg, b1r, W2)
    acc2 = segsum_kernel(y2, src_r, dst_r, zeros_r)
    out = _epi_final(acc2, y2, deg, b2r)
    return out[:N]


# speedup vs baseline: 1.0009x; 1.0009x over previous
import dataclasses
import functools

import jax
import jax.numpy as jnp
from jax import lax
from jax.experimental import pallas as pl
from jax.experimental.pallas import tpu as pltpu
from jax.experimental.pallas import tpu_sc as plsc

N = 10000
D = 128
E = 320000

NC = 2
NS = 16
NW = NC * NS

CH = 128
NCHUNK = 80
EPT = NCHUNK * CH
E_PAD = NW * EPT
NPAD = 10240
RPT = NPAD // NS
SCH = 64
SNCHUNK = EPT // SCH
NBUF = 4
SNPASS = 4
SCPP = SNCHUNK // SNPASS

RB = 512
GRID = NPAD // RB


@functools.cache
def _sc_kernels():
    mesh = plsc.VectorSubcoreMesh(core_axis_name="c", subcore_axis_name="s")
    cp = pltpu.CompilerParams()
    if "needs_layout_passes" in pltpu.CompilerParams.__dataclass_fields__:
        cp = dataclasses.replace(cp, needs_layout_passes=False)

    L = 16
    HR = NPAD // D

    @functools.partial(
        pl.kernel,
        out_type=jax.ShapeDtypeStruct((NC, HR, D), jnp.float32),
        mesh=mesh,
        compiler_params=cp,
        scratch_types=[
            pltpu.VMEM((NCHUNK, CH), jnp.int32),
            pltpu.VMEM((HR, D), jnp.float32),
            pltpu.VMEM((HR,), jnp.int32),
            pltpu.VMEM_SHARED((HR, D), jnp.float32),
        ],
    )
    def deg_kernel(dst_hbm, zeros_hbm, rowid_hbm, out_hbm,
                   dst_v, hist_v, rowid_v, deg_sh):
        cid = lax.axis_index("c")
        sid = lax.axis_index("s")
        wid = sid * NC + cid
        pltpu.sync_copy(dst_hbm.at[wid], dst_v)
        pltpu.sync_copy(zeros_hbm, hist_v)
        pltpu.sync_copy(rowid_hbm, rowid_v)

        @pl.when(sid == 0)
        def _():
            pltpu.sync_copy(zeros_hbm, deg_sh)

        ones = jnp.ones((L,), jnp.float32)

        @pl.loop(0, NCHUNK)
        def _(j):
            @pl.loop(0, CH, step=L)
            def _(k):
                idx = dst_v.at[j][pl.ds(k, L)]
                hi = jnp.right_shift(idx, 7)
                lo = jnp.bitwise_and(idx, 127)
                plsc.addupdate_scatter(hist_v, [hi, lo], ones)

        plsc.subcore_barrier()
        pltpu.sync_copy(hist_v, deg_sh.at[rowid_v], add=True)
        plsc.subcore_barrier()

        @pl.when(sid == 0)
        def _():
            pltpu.sync_copy(deg_sh, out_hbm.at[cid])

    @functools.partial(
        pl.kernel,
        out_type=jax.ShapeDtypeStruct((NC, NPAD, D), jnp.float32),
        mesh=mesh,
        scratch_types=[
            pltpu.VMEM((SCPP, SCH), jnp.int32),
            pltpu.VMEM((SCPP, SCH), jnp.int32),
        ] + [pltpu.VMEM((SCH, D), jnp.float32)] * NBUF + [
            pltpu.VMEM_SHARED((NPAD, D), jnp.float32),
        ] + [pltpu.SemaphoreType.DMA] * (2 * NBUF),
    )
    def segsum_kernel(y_hbm, src_hbm, dst_hbm, zeros_hbm, out_hbm,
                      src_v, dst_v, *rest):
        rows = rest[:NBUF]
        acc_sh = rest[NBUF]
        gsem = rest[NBUF + 1:NBUF + 1 + NBUF]
        ssem = rest[NBUF + 1 + NBUF:]
        cid = lax.axis_index("c")
        sid = lax.axis_index("s")
        wid = sid * NC + cid
        pltpu.sync_copy(zeros_hbm, acc_sh.at[pl.ds(sid * RPT, RPT)])
        plsc.subcore_barrier()

        def gather(j, b):
            return pltpu.async_copy(y_hbm.at[src_v.at[j]], rows[b], gsem[b])

        def gather_wait(j, b):
            pltpu.make_async_copy(y_hbm.at[src_v.at[j]], rows[b],
                                  gsem[b]).wait()

        def scatter(j, b):
            return pltpu.async_copy(rows[b], acc_sh.at[dst_v.at[j]], ssem[b],
                                    add=True)

        def scatter_wait(j, b):
            pltpu.make_async_copy(rows[b], acc_sh.at[dst_v.at[j]],
                                  ssem[b]).wait()

        for p in range(SNPASS):
            pltpu.sync_copy(src_hbm.at[wid, pl.ds(p * SCPP, SCPP)], src_v)
            pltpu.sync_copy(dst_hbm.at[wid, pl.ds(p * SCPP, SCPP)], dst_v)
            for b in range(NBUF):
                gather(b, b)

            @pl.loop(0, SCPP, step=NBUF)
            def _(j):
                for b in range(NBUF):
                    gather_wait(j + b, b)
                    scatter(j + b, b)
                for b in range(NBUF):
                    scatter_wait(j + b, b)

                    @pl.when(j + b + NBUF < SCPP)
                    def _():
                        gather(j + b + NBUF, b)

        plsc.subcore_barrier()
        pltpu.sync_copy(
            acc_sh.at[pl.ds(sid * RPT, RPT)],
            out_hbm.at[cid, pl.ds(sid * RPT, RPT)],
        )

    return deg_kernel, segsum_kernel


def _dinv_of(deg_ref):
    deg = deg_ref[0, :, 0] + deg_ref[1, :, 0] + 1.0
    return lax.rsqrt(deg)


def _mm_scale_body(x_ref, w_ref, deg_ref, y_ref):
    dinv = _dinv_of(deg_ref)
    xw = jnp.dot(x_ref[...], w_ref[...], preferred_element_type=jnp.float32,
                 precision=lax.Precision.HIGHEST)
    y_ref[...] = dinv[:, None] * xw


def _epi_mm_body(acc_ref, y_ref, deg_ref, b_ref, w_ref, out_ref):
    dinv = _dinv_of(deg_ref)
    s = acc_ref[0] + acc_ref[1] + y_ref[...]
    z = jnp.maximum(dinv[:, None] * s + b_ref[0], 0.0)
    zw = jnp.dot(z, w_ref[...], preferred_element_type=jnp.float32,
                 precision=lax.Precision.HIGHEST)
    out_ref[...] = dinv[:, None] * zw


def _epi_final_body(acc_ref, y_ref, deg_ref, b_ref, out_ref):
    dinv = _dinv_of(deg_ref)
    s = acc_ref[0] + acc_ref[1] + y_ref[...]
    out_ref[...] = jnp.maximum(dinv[:, None] * s + b_ref[0], 0.0)


_spec_rows = pl.BlockSpec((RB, D), lambda i: (i, 0))
_spec_w = pl.BlockSpec((D, D), lambda i: (0, 0))
_spec_deg = pl.BlockSpec((NC, RB, 1), lambda i: (0, i, 0))
_spec_acc = pl.BlockSpec((NC, RB, D), lambda i: (0, i, 0))
_spec_b = pl.BlockSpec((1, D), lambda i: (0, 0))
_out_rows = jax.ShapeDtypeStruct((NPAD, D), jnp.float32)

_mm_scale = pl.pallas_call(
    _mm_scale_body, grid=(GRID,),
    in_specs=[_spec_rows, _spec_w, _spec_deg],
    out_specs=_spec_rows, out_shape=_out_rows)

_epi_mm = pl.pallas_call(
    _epi_mm_body, grid=(GRID,),
    in_specs=[_spec_acc, _spec_rows, _spec_deg, _spec_b, _spec_w],
    out_specs=_spec_rows, out_shape=_out_rows)

_epi_final = pl.pallas_call(
    _epi_final_body, grid=(GRID,),
    in_specs=[_spec_acc, _spec_rows, _spec_deg, _spec_b],
    out_specs=_spec_rows, out_shape=_out_rows)


def kernel(x, edge_index, W1, b1, W2, b2):
    src = edge_index[0].astype(jnp.int32)
    dst = edge_index[1].astype(jnp.int32)
    ppt = EPT - E // NW
    pad_src = jnp.zeros((NW, ppt), jnp.int32)
    pad_dst = jnp.broadcast_to(N + jnp.arange(ppt, dtype=jnp.int32) % (NPAD - N),
                               (NW, ppt))
    src_t = jnp.concatenate([src.reshape(NW, E // NW), pad_src], axis=1)
    dst_t = jnp.concatenate([dst.reshape(NW, E // NW), pad_dst], axis=1)
    src_r = src_t.reshape(NW, SNCHUNK, SCH)
    dst_r = dst_t.reshape(NW, SNCHUNK, SCH)
    dst_d = dst_t.reshape(NW, NCHUNK, CH)

    x_p = jnp.pad(x, ((0, NPAD - N), (0, 0)))
    zeros_h = jnp.zeros((NPAD // D, D), jnp.float32)
    rowid = jnp.arange(NPAD // D, dtype=jnp.int32)
    zeros_r = jnp.zeros((RPT, D), jnp.float32)
    b1r = b1.reshape(1, D)
    b2r = b2.reshape(1, D)

    deg_kernel, segsum_kernel = _sc_kernels()
    deg = deg_kernel(dst_d, zeros_h, rowid).reshape(NC, NPAD, 1)

    y1 = _mm_scale(x_p, W1, deg)
    acc1 = segsum_kernel(y1, src_r, dst_r, zeros_r)
    y2 = _epi_mm(acc1, y1, deg, b1r, W2)
    acc2 = segsum_kernel(y2, src_r, dst_r, zeros_r)
    out = _epi_final(acc2, y2, deg, b2r)
    return out[:N]
